# baseline (speedup 1.0000x reference)
import functools

import jax
from jax import lax
from jax.experimental import pallas as pl
from jax.experimental.pallas import tpu as pltpu
from jax.experimental.pallas import tpu_sc as plsc

_NUM_CORES = 2
_NUM_SUBCORES = 16
_NUM_WORKERS = _NUM_CORES * _NUM_SUBCORES


def _make_sc_broadcast(S, B, D, dtype):
    rows_per_worker = S // _NUM_WORKERS
    chunk = 8
    nchunk = rows_per_worker // chunk
    mesh = plsc.VectorSubcoreMesh(core_axis_name="c", subcore_axis_name="s")

    @functools.partial(
        pl.kernel,
        mesh=mesh,
        out_type=jax.ShapeDtypeStruct((S, B, D), dtype),
        scratch_types=[
            pltpu.VMEM((chunk, D), dtype),
            pltpu.VMEM((chunk, D), dtype),
            pltpu.SemaphoreType.DMA,
            pltpu.SemaphoreType.DMA,
        ],
    )
    def sc_broadcast(table_hbm, out_hbm, buf0, buf1, gsem, ssem):
        wid = lax.axis_index("s") * _NUM_CORES + lax.axis_index("c")
        base = wid * rows_per_worker
        bufs = (buf0, buf1)
        gathers = [None] * nchunk
        scatters = [[] for _ in range(nchunk)]
        gathers[0] = pltpu.async_copy(table_hbm.at[pl.ds(base, chunk)], bufs[0], gsem)
        gathers[1] = pltpu.async_copy(
            table_hbm.at[pl.ds(base + chunk, chunk)], bufs[1], gsem
        )
        for i in range(nchunk):
            gathers[i].wait()
            for b in range(B):
                scatters[i].append(
                    pltpu.async_copy(
                        bufs[i % 2],
                        out_hbm.at[pl.ds(base + i * chunk, chunk), b],
                        ssem,
                    )
                )
            if i + 2 < nchunk:
                for c in scatters[i]:
                    c.wait()
                gathers[i + 2] = pltpu.async_copy(
                    table_hbm.at[pl.ds(base + (i + 2) * chunk, chunk)],
                    bufs[i % 2],
                    gsem,
                )
        for c in scatters[nchunk - 2] + scatters[nchunk - 1]:
            c.wait()

    return sc_broadcast


def kernel(x, pos_embed_weight):
    S, B, _ = x.shape
    _, D = pos_embed_weight.shape
    fn = _make_sc_broadcast(S, B, D, pos_embed_weight.dtype)
    return fn(pos_embed_weight[:S])


# speedup vs baseline: 1.0814x; 1.0814x over previous
import functools

import jax
from jax import lax
from jax.experimental import pallas as pl
from jax.experimental.pallas import tpu as pltpu
from jax.experimental.pallas import tpu_sc as plsc

_NUM_CORES = 2
_NUM_SUBCORES = 16
_NUM_WORKERS = _NUM_CORES * _NUM_SUBCORES


def _make_sc_broadcast(S, B, D, dtype):
    rows_per_worker = S // _NUM_WORKERS
    chunk = 16
    nchunk = rows_per_worker // chunk
    mesh = plsc.VectorSubcoreMesh(core_axis_name="c", subcore_axis_name="s")

    @functools.partial(
        pl.kernel,
        mesh=mesh,
        out_type=jax.ShapeDtypeStruct((S, B, D), dtype),
        scratch_types=[
            pltpu.VMEM((chunk, D), dtype),
            pltpu.VMEM((chunk, D), dtype),
            pltpu.SemaphoreType.DMA,
            pltpu.SemaphoreType.DMA,
        ],
    )
    def sc_broadcast(table_hbm, out_hbm, buf0, buf1, gsem, ssem):
        wid = lax.axis_index("s") * _NUM_CORES + lax.axis_index("c")
        base = wid * rows_per_worker
        bufs = (buf0, buf1)
        gathers = [None] * nchunk
        scatters = [[] for _ in range(nchunk)]
        gathers[0] = pltpu.async_copy(table_hbm.at[pl.ds(base, chunk)], bufs[0], gsem)
        gathers[1] = pltpu.async_copy(
            table_hbm.at[pl.ds(base + chunk, chunk)], bufs[1], gsem
        )
        for i in range(nchunk):
            gathers[i].wait()
            for b in range(B):
                scatters[i].append(
                    pltpu.async_copy(
                        bufs[i % 2],
                        out_hbm.at[pl.ds(base + i * chunk, chunk), b],
                        ssem,
                    )
                )
            if i + 2 < nchunk:
                for c in scatters[i]:
                    c.wait()
                gathers[i + 2] = pltpu.async_copy(
                    table_hbm.at[pl.ds(base + (i + 2) * chunk, chunk)],
                    bufs[i % 2],
                    gsem,
                )
        for c in scatters[nchunk - 2] + scatters[nchunk - 1]:
            c.wait()

    return sc_broadcast


def kernel(x, pos_embed_weight):
    S, B, _ = x.shape
    _, D = pos_embed_weight.shape
    fn = _make_sc_broadcast(S, B, D, pos_embed_weight.dtype)
    return fn(pos_embed_weight[:S])


# speedup vs baseline: 1.0845x; 1.0028x over previous
import functools

import jax
from jax import lax
from jax.experimental import pallas as pl
from jax.experimental.pallas import tpu as pltpu
from jax.experimental.pallas import tpu_sc as plsc

_NUM_CORES = 2
_NUM_SUBCORES = 16
_NUM_WORKERS = _NUM_CORES * _NUM_SUBCORES


def _make_sc_broadcast(S, B, D, dtype):
    rows_per_worker = S // _NUM_WORKERS
    chunk = 16
    nchunk = rows_per_worker // chunk
    mesh = plsc.VectorSubcoreMesh(core_axis_name="c", subcore_axis_name="s")

    @functools.partial(
        pl.kernel,
        mesh=mesh,
        out_type=jax.ShapeDtypeStruct((S, B, D), dtype),
        scratch_types=[
            pltpu.VMEM((chunk, D), dtype),
            pltpu.VMEM((chunk, D), dtype),
            pltpu.VMEM((chunk, D), dtype),
            pltpu.SemaphoreType.DMA,
            pltpu.SemaphoreType.DMA,
        ],
    )
    def sc_broadcast(table_hbm, out_hbm, buf0, buf1, buf2, gsem, ssem):
        wid = lax.axis_index("s") * _NUM_CORES + lax.axis_index("c")
        base = wid * rows_per_worker
        bufs = (buf0, buf1, buf2)
        nbuf = len(bufs)
        gathers = [None] * nchunk
        scatters = [[] for _ in range(nchunk)]
        for j in range(nbuf):
            gathers[j] = pltpu.async_copy(
                table_hbm.at[pl.ds(base + j * chunk, chunk)], bufs[j], gsem
            )
        for i in range(nchunk):
            gathers[i].wait()
            for b in range(B):
                scatters[i].append(
                    pltpu.async_copy(
                        bufs[i % nbuf],
                        out_hbm.at[pl.ds(base + i * chunk, chunk), b],
                        ssem,
                    )
                )
            if i >= 1 and i + 2 < nchunk:
                for c in scatters[i - 1]:
                    c.wait()
                gathers[i + 2] = pltpu.async_copy(
                    table_hbm.at[pl.ds(base + (i + 2) * chunk, chunk)],
                    bufs[(i + 2) % nbuf],
                    gsem,
                )
        for i in range(max(0, nchunk - 3), nchunk):
            for c in scatters[i]:
                c.wait()

    return sc_broadcast


def kernel(x, pos_embed_weight):
    S, B, _ = x.shape
    _, D = pos_embed_weight.shape
    fn = _make_sc_broadcast(S, B, D, pos_embed_weight.dtype)
    return fn(pos_embed_weight[:S])
